# baseline (speedup 1.0000x reference)
import functools

import jax
import jax.numpy as jnp
from jax import lax
from jax.experimental import pallas as pl
from jax.experimental.pallas import tpu as pltpu
from jax.experimental.pallas import tpu_sc as plsc

N_NODES = 10000
N_PAD = 10112
D = 128
NUM_GRAPHS = 64
E_EDGES = 320000
NUM_WORKERS = 32
CHUNK = 128
CHUNKS = 80
E_PAD = NUM_WORKERS * CHUNKS * CHUNK
ROWS_PER_TILE = N_PAD // 16

_HIGH = jax.lax.Precision.HIGHEST


def _sc_mesh():
    return plsc.VectorSubcoreMesh(core_axis_name="c", subcore_axis_name="s")


@functools.partial(
    pl.kernel,
    out_type=jax.ShapeDtypeStruct((2, N_PAD, D), jnp.float32),
    mesh=_sc_mesh(),
    scratch_types=[
        pltpu.VMEM((CHUNKS, CHUNK), jnp.int32),
        pltpu.VMEM((CHUNK, D), jnp.float32),
        pltpu.VMEM_SHARED((N_PAD, D), jnp.float32),
    ],
)
def _deg_sc(dst_hbm, ones_hbm, zeros_hbm, out_hbm, dst_v, ones_v, acc):
    c = lax.axis_index("c")
    s = lax.axis_index("s")
    w = c * 16 + s
    pltpu.sync_copy(dst_hbm.at[w], dst_v)
    pltpu.sync_copy(ones_hbm, ones_v)
    sl = pl.ds(s * ROWS_PER_TILE, ROWS_PER_TILE)
    pltpu.sync_copy(zeros_hbm.at[sl], acc.at[sl])
    plsc.subcore_barrier()

    def body(j, carry):
        pltpu.sync_copy(ones_v, acc.at[dst_v.at[j]], add=True)
        return carry

    lax.fori_loop(0, CHUNKS, body, 0)
    plsc.subcore_barrier()
    pltpu.sync_copy(acc.at[sl], out_hbm.at[c, sl])


@functools.partial(
    pl.kernel,
    out_type=jax.ShapeDtypeStruct((2, N_PAD, D), jnp.float32),
    mesh=_sc_mesh(),
    scratch_types=[
        pltpu.VMEM((CHUNKS, CHUNK), jnp.int32),
        pltpu.VMEM((CHUNKS, CHUNK), jnp.int32),
        pltpu.VMEM((CHUNK, D), jnp.float32),
        pltpu.VMEM_SHARED((N_PAD, D), jnp.float32),
        pltpu.SemaphoreType.DMA,
    ],
)
def _scatter_sc(h_hbm, src_hbm, dst_hbm, out_hbm, src_v, dst_v, rows_v, acc,
                sem):
    c = lax.axis_index("c")
    s = lax.axis_index("s")
    w = c * 16 + s
    pltpu.sync_copy(src_hbm.at[w], src_v)
    pltpu.sync_copy(dst_hbm.at[w], dst_v)
    sl = pl.ds(s * ROWS_PER_TILE, ROWS_PER_TILE)
    pltpu.sync_copy(h_hbm.at[sl], acc.at[sl])
    plsc.subcore_barrier()

    def body(j, carry):
        pltpu.async_copy(h_hbm.at[src_v.at[j]], rows_v, sem).wait()
        pltpu.sync_copy(rows_v, acc.at[dst_v.at[j]], add=True)
        return carry

    lax.fori_loop(0, CHUNKS, body, 0)
    plsc.subcore_barrier()
    pltpu.sync_copy(acc.at[sl], out_hbm.at[c, sl])


def _row_mask():
    rows = lax.broadcasted_iota(jnp.int32, (N_PAD, 1), 0)
    return (rows < N_NODES).astype(jnp.float32)


def _prep_body(x_ref, w_ref, deg_ref, h_ref, dinv_ref):
    dinv = lax.rsqrt(jnp.maximum(deg_ref[0] + deg_ref[1] + 1.0, 1.0))
    h = jnp.dot(x_ref[...], w_ref[...], precision=_HIGH,
                preferred_element_type=jnp.float32)
    h_ref[...] = h * dinv * _row_mask()
    dinv_ref[...] = dinv


def _mid_body(acc_ref, hp_ref, dinv_ref, b_ref, g_ref, be_ref, w_ref, out_ref):
    dinv = dinv_ref[...]
    z = (acc_ref[0] + acc_ref[1] - hp_ref[...]) * dinv + b_ref[...]
    zq = z[:N_NODES]
    m = jnp.mean(zq, axis=0, keepdims=True)
    v = jnp.mean((zq - m) ** 2, axis=0, keepdims=True)
    a = jnp.maximum((zq - m) * lax.rsqrt(v + 1e-5) * g_ref[...] + be_ref[...],
                    0.0)
    h = jnp.dot(a, w_ref[...], precision=_HIGH,
                preferred_element_type=jnp.float32) * dinv[:N_NODES]
    out_ref[pl.ds(0, N_NODES), :] = h
    out_ref[pl.ds(N_NODES, N_PAD - N_NODES), :] = jnp.zeros(
        (N_PAD - N_NODES, D), jnp.float32)


def _final_body(acc_ref, hp_ref, dinv_ref, b_ref, batch_ref, out_ref):
    dinv = dinv_ref[...]
    z = (acc_ref[0] + acc_ref[1] - hp_ref[...]) * dinv + b_ref[...]
    zq = z[:N_NODES]
    gid = lax.broadcasted_iota(jnp.int32, (N_NODES, NUM_GRAPHS), 1)
    onehot = (batch_ref[...] == gid).astype(jnp.float32)
    out_ref[...] = lax.dot_general(
        onehot, zq, (((0,), (0,)), ((), ())), precision=_HIGH,
        preferred_element_type=jnp.float32)


_prep_tc = pl.pallas_call(
    _prep_body, out_shape=[jax.ShapeDtypeStruct((N_PAD, D), jnp.float32),
                           jax.ShapeDtypeStruct((N_PAD, D), jnp.float32)])

_mid_tc = pl.pallas_call(
    _mid_body, out_shape=jax.ShapeDtypeStruct((N_PAD, D), jnp.float32))

_final_tc = pl.pallas_call(
    _final_body, out_shape=jax.ShapeDtypeStruct((NUM_GRAPHS, D), jnp.float32))


def kernel(x, edge_index, batch, W1, b1, g1, be1, W2, b2, g2, be2, W3, b3):
    pad = E_PAD - E_EDGES
    fill = jnp.full((pad,), N_NODES, jnp.int32)
    src3 = jnp.concatenate([edge_index[0], fill]).reshape(
        NUM_WORKERS, CHUNKS, CHUNK)
    dst3 = jnp.concatenate([edge_index[1], fill]).reshape(
        NUM_WORKERS, CHUNKS, CHUNK)
    xp = jnp.concatenate(
        [x, jnp.zeros((N_PAD - N_NODES, D), jnp.float32)], axis=0)
    ones_wide = jnp.ones((CHUNK, D), jnp.float32)
    zeros_wide = jnp.zeros((N_PAD, D), jnp.float32)

    deg = _deg_sc(dst3, ones_wide, zeros_wide)
    h1, dinv = _prep_tc(xp, W1, deg)
    acc1 = _scatter_sc(h1, src3, dst3)
    h2 = _mid_tc(acc1, h1, dinv, b1.reshape(1, D), g1.reshape(1, D),
                 be1.reshape(1, D), W2)
    acc2 = _scatter_sc(h2, src3, dst3)
    h3 = _mid_tc(acc2, h2, dinv, b2.reshape(1, D), g2.reshape(1, D),
                 be2.reshape(1, D), W3)
    acc3 = _scatter_sc(h3, src3, dst3)
    return _final_tc(acc3, h3, dinv, b3.reshape(1, D),
                     batch.reshape(N_NODES, 1))


# speedup vs baseline: 2.2695x; 2.2695x over previous
import functools

import jax
import jax.numpy as jnp
from jax import lax
from jax.experimental import pallas as pl
from jax.experimental.pallas import tpu as pltpu
from jax.experimental.pallas import tpu_sc as plsc

N_NODES = 10000
N_PAD = 10112
D = 128
NUM_GRAPHS = 64
E_EDGES = 320000
NUM_WORKERS = 32
CHUNK = 128
CHUNKS = 79
E_PAD = NUM_WORKERS * CHUNKS * CHUNK
ROWS_PER_TILE = N_PAD // 16

_HIGH = jax.lax.Precision.HIGHEST


def _sc_mesh():
    return plsc.VectorSubcoreMesh(core_axis_name="c", subcore_axis_name="s")


@functools.partial(
    pl.kernel,
    out_type=jax.ShapeDtypeStruct((2, N_PAD, D), jnp.float32),
    mesh=_sc_mesh(),
    scratch_types=[
        pltpu.VMEM((CHUNKS, CHUNK), jnp.int32),
        pltpu.VMEM((CHUNK, D), jnp.float32),
        pltpu.VMEM_SHARED((N_PAD, D), jnp.float32),
    ],
)
def _deg_sc(dst_hbm, ones_hbm, zeros_hbm, out_hbm, dst_v, ones_v, acc):
    c = lax.axis_index("c")
    s = lax.axis_index("s")
    w = c * 16 + s
    pltpu.sync_copy(dst_hbm.at[w], dst_v)
    pltpu.sync_copy(ones_hbm, ones_v)
    sl = pl.ds(s * ROWS_PER_TILE, ROWS_PER_TILE)
    pltpu.sync_copy(zeros_hbm.at[sl], acc.at[sl])
    plsc.subcore_barrier()

    def body(j, carry):
        pltpu.sync_copy(ones_v, acc.at[dst_v.at[j]], add=True)
        return carry

    lax.fori_loop(0, CHUNKS, body, 0)
    plsc.subcore_barrier()
    pltpu.sync_copy(acc.at[sl], out_hbm.at[c, sl])


@functools.partial(
    pl.kernel,
    out_type=jax.ShapeDtypeStruct((2, N_PAD, D), jnp.float32),
    mesh=_sc_mesh(),
    scratch_types=[
        pltpu.VMEM((CHUNKS, CHUNK), jnp.int32),
        pltpu.VMEM((CHUNKS, CHUNK), jnp.int32),
        pltpu.VMEM((CHUNK, D), jnp.float32),
        pltpu.VMEM_SHARED((N_PAD, D), jnp.float32),
        pltpu.SemaphoreType.DMA,
    ],
)
def _scatter_sc(h_hbm, src_hbm, dst_hbm, out_hbm, src_v, dst_v, rows_v, acc,
                sem):
    c = lax.axis_index("c")
    s = lax.axis_index("s")
    w = c * 16 + s
    pltpu.sync_copy(src_hbm.at[w], src_v)
    pltpu.sync_copy(dst_hbm.at[w], dst_v)
    sl = pl.ds(s * ROWS_PER_TILE, ROWS_PER_TILE)
    pltpu.sync_copy(h_hbm.at[sl], acc.at[sl])
    plsc.subcore_barrier()

    def body(j, carry):
        pltpu.async_copy(h_hbm.at[src_v.at[j]], rows_v, sem).wait()
        pltpu.sync_copy(rows_v, acc.at[dst_v.at[j]], add=True)
        return carry

    lax.fori_loop(0, CHUNKS, body, 0)
    plsc.subcore_barrier()
    pltpu.sync_copy(acc.at[sl], out_hbm.at[c, sl])


def _row_mask():
    rows = lax.broadcasted_iota(jnp.int32, (N_PAD, 1), 0)
    return (rows < N_NODES).astype(jnp.float32)


def _prep_body(x_ref, w_ref, deg_ref, h_ref, dinv_ref):
    dinv = lax.rsqrt(jnp.maximum(deg_ref[0] + deg_ref[1] + 1.0, 1.0))
    h = jnp.dot(x_ref[...], w_ref[...], precision=_HIGH,
                preferred_element_type=jnp.float32)
    h_ref[...] = h * dinv * _row_mask()
    dinv_ref[...] = dinv


def _mid_body(acc_ref, hp_ref, dinv_ref, b_ref, g_ref, be_ref, w_ref, out_ref):
    dinv = dinv_ref[...]
    z = (acc_ref[0] + acc_ref[1] - hp_ref[...]) * dinv + b_ref[...]
    zq = z[:N_NODES]
    m = jnp.mean(zq, axis=0, keepdims=True)
    v = jnp.mean((zq - m) ** 2, axis=0, keepdims=True)
    a = jnp.maximum((zq - m) * lax.rsqrt(v + 1e-5) * g_ref[...] + be_ref[...],
                    0.0)
    h = jnp.dot(a, w_ref[...], precision=_HIGH,
                preferred_element_type=jnp.float32) * dinv[:N_NODES]
    out_ref[pl.ds(0, N_NODES), :] = h
    out_ref[pl.ds(N_NODES, N_PAD - N_NODES), :] = jnp.zeros(
        (N_PAD - N_NODES, D), jnp.float32)


def _final_body(acc_ref, hp_ref, dinv_ref, b_ref, batch_ref, out_ref):
    dinv = dinv_ref[...]
    z = (acc_ref[0] + acc_ref[1] - hp_ref[...]) * dinv + b_ref[...]
    zq = z[:N_NODES]
    gid = lax.broadcasted_iota(jnp.int32, (N_NODES, NUM_GRAPHS), 1)
    onehot = (batch_ref[...] == gid).astype(jnp.float32)
    out_ref[...] = lax.dot_general(
        onehot, zq, (((0,), (0,)), ((), ())), precision=_HIGH,
        preferred_element_type=jnp.float32)


_prep_tc = pl.pallas_call(
    _prep_body, out_shape=[jax.ShapeDtypeStruct((N_PAD, D), jnp.float32),
                           jax.ShapeDtypeStruct((N_PAD, D), jnp.float32)])

_mid_tc = pl.pallas_call(
    _mid_body, out_shape=jax.ShapeDtypeStruct((N_PAD, D), jnp.float32))

_final_tc = pl.pallas_call(
    _final_body, out_shape=jax.ShapeDtypeStruct((NUM_GRAPHS, D), jnp.float32))


def kernel(x, edge_index, batch, W1, b1, g1, be1, W2, b2, g2, be2, W3, b3):
    pad = E_PAD - E_EDGES
    fill = jnp.full((pad,), N_NODES, jnp.int32)
    src3 = jnp.concatenate([edge_index[0], fill]).reshape(
        NUM_WORKERS, CHUNKS, CHUNK)
    dst3 = jnp.concatenate([edge_index[1], fill]).reshape(
        NUM_WORKERS, CHUNKS, CHUNK)
    xp = jnp.concatenate(
        [x, jnp.zeros((N_PAD - N_NODES, D), jnp.float32)], axis=0)
    ones_wide = jnp.ones((CHUNK, D), jnp.float32)
    zeros_wide = jnp.zeros((N_PAD, D), jnp.float32)

    deg = _deg_sc(dst3, ones_wide, zeros_wide)
    h1, dinv = _prep_tc(xp, W1, deg)
    acc1 = _scatter_sc(h1, src3, dst3)
    h2 = _mid_tc(acc1, h1, dinv, b1.reshape(1, D), g1.reshape(1, D),
                 be1.reshape(1, D), W2)
    acc2 = _scatter_sc(h2, src3, dst3)
    h3 = _mid_tc(acc2, h2, dinv, b2.reshape(1, D), g2.reshape(1, D),
                 be2.reshape(1, D), W3)
    acc3 = _scatter_sc(h3, src3, dst3)
    return _final_tc(acc3, h3, dinv, b3.reshape(1, D),
                     batch.reshape(N_NODES, 1))
